# baseline (speedup 1.0000x reference)
import jax
import jax.numpy as jnp
from jax import lax
from jax.experimental import pallas as pl

_CN = 1000


def _score_body(lg_ref, obj_ref, unk_ref, qs_ref, ql_ref):
    lg = lg_ref[0]
    obj = obj_ref[0]
    unk = unk_ref[0]

    obj_prob = jnp.exp(-obj)
    kp = jax.nn.sigmoid(lg)
    cls = lax.broadcasted_iota(jnp.int32, kp.shape, 1)
    inv = (cls >= 75) & (cls <= 79)
    kp = jnp.where(inv | (cls == 90), 0.0, kp)
    up = jax.nn.sigmoid(unk)
    sup = jnp.clip(1.0 - up, 0.0, 1.0)
    ks = (obj_prob * kp) * sup
    ks = jnp.where(inv | (cls >= 90), -1.0, ks)
    best = jnp.max(ks, axis=-1, keepdims=True)
    lbl = jnp.argmax(ks, axis=-1).astype(jnp.int32)[:, None]
    best = jnp.maximum(best, 0.0)
    us = obj_prob * up
    choose = us >= jnp.float32(0.95) * best
    qs_ref[0] = jnp.where(choose, us, best)
    ql_ref[0] = jnp.where(choose, 90, lbl)


def _query_scores(pred_logits, pred_obj, pred_unk):
    B, N, C = pred_logits.shape
    grid = (B, N // _CN)
    qs, ql = pl.pallas_call(
        _score_body,
        grid=grid,
        in_specs=[
            pl.BlockSpec((1, _CN, C), lambda b, n: (b, n, 0)),
            pl.BlockSpec((1, _CN, 1), lambda b, n: (b, n, 0)),
            pl.BlockSpec((1, _CN, 1), lambda b, n: (b, n, 0)),
        ],
        out_specs=[
            pl.BlockSpec((1, _CN, 1), lambda b, n: (b, n, 0)),
            pl.BlockSpec((1, _CN, 1), lambda b, n: (b, n, 0)),
        ],
        out_shape=[
            jax.ShapeDtypeStruct((B, N, 1), jnp.float32),
            jax.ShapeDtypeStruct((B, N, 1), jnp.int32),
        ],
    )(pred_logits, pred_obj[..., None], pred_unk[..., None])
    return qs[..., 0], ql[..., 0]


def kernel(pred_logits, pred_obj, pred_unk, pred_boxes, target_sizes):
    qs, ql = _query_scores(pred_logits, pred_obj, pred_unk)
    scores, topk_idx = lax.top_k(qs, 100)
    labels = jnp.take_along_axis(ql, topk_idx, axis=1)
    xc, yc, w, h = (pred_boxes[..., 0], pred_boxes[..., 1],
                    pred_boxes[..., 2], pred_boxes[..., 3])
    boxes = jnp.stack([xc - 0.5 * w, yc - 0.5 * h,
                       xc + 0.5 * w, yc + 0.5 * h], axis=-1)
    boxes = jnp.take_along_axis(boxes, topk_idx[..., None], axis=1)
    img_h, img_w = target_sizes[:, 0], target_sizes[:, 1]
    scale_fct = jnp.stack([img_w, img_h, img_w, img_h], axis=1)
    boxes = boxes * scale_fct[:, None, :]
    return scores, labels, boxes
